# baseline (speedup 1.0000x reference)
import functools

import jax
import jax.numpy as jnp
from jax import lax
from jax.experimental import pallas as pl
from jax.experimental.pallas import tpu as pltpu
from jax.experimental.pallas import tpu_sc as plsc

N = 16384
E = 64
K = 2
NC = 2
NS = 16
LANES = 16
ROWS_PER_TILE = N // (NC * NS)
SEL_PER_TILE = (N * K) // NS
SCALE = float(E) / (float(N) * K * N)
ECH = E // LANES
NCHUNK = 4
CHUNK_ROWS = ROWS_PER_TILE // NCHUNK


_mesh = plsc.VectorSubcoreMesh(
    core_axis_name="c", subcore_axis_name="s", num_cores=NC, num_subcores=NS
)


@functools.partial(
    pl.kernel,
    out_type=jax.ShapeDtypeStruct((NC, LANES), jnp.float32),
    mesh=_mesh,
    compiler_params=pltpu.CompilerParams(needs_layout_passes=False),
    scratch_types=[
        pltpu.VMEM((CHUNK_ROWS * E,), jnp.float32),
        pltpu.VMEM((CHUNK_ROWS * E,), jnp.float32),
        pltpu.VMEM((SEL_PER_TILE,), jnp.int32),
        pltpu.VMEM((E,), jnp.float32),
        pltpu.VMEM((E,), jnp.int32),
        pltpu.VMEM((LANES,), jnp.float32),
        pltpu.VMEM_SHARED((E,), jnp.float32),
        pltpu.VMEM_SHARED((LANES,), jnp.float32),
        pltpu.SemaphoreType.DMA,
        pltpu.SemaphoreType.DMA,
        pltpu.SemaphoreType.DMA,
    ],
)
def _lb_loss_kernel(probs_hbm, sel_hbm, out_hbm,
                    buf_a, buf_b, sel_v, col_v, idx_v, vec_v,
                    shared_col, shared_acc, sem_a, sem_b, sem_sel):
    c = lax.axis_index("c")
    s = lax.axis_index("s")
    iota16 = lax.iota(jnp.int32, LANES)
    for j in range(ECH):
        idx_v[pl.ds(j * LANES, LANES)] = iota16 + j * LANES

    sel_cp = pltpu.async_copy(
        sel_hbm.at[pl.ds(s * SEL_PER_TILE, SEL_PER_TILE)], sel_v, sem_sel)

    @pl.when(s == 0)
    def _zero_shared():
        for j in range(ECH):
            col_v[pl.ds(j * LANES, LANES)] = jnp.zeros((LANES,), jnp.float32)
        vec_v[...] = jnp.zeros((LANES,), jnp.float32)
        pltpu.sync_copy(col_v, shared_col)
        pltpu.sync_copy(vec_v, shared_acc)

    bufs = (buf_a, buf_b)
    sems = (sem_a, sem_b)
    row_base = (c * NS + s) * (ROWS_PER_TILE * E)

    def start_chunk(ci):
        return pltpu.async_copy(
            probs_hbm.at[pl.ds(row_base + ci * (CHUNK_ROWS * E),
                               CHUNK_ROWS * E)],
            bufs[ci % 2], sems[ci % 2])

    copies = [start_chunk(0)]
    accs = tuple(jnp.zeros((LANES,), jnp.float32) for _ in range(ECH))
    for ci in range(NCHUNK):
        copies[ci].wait()
        if ci + 1 < NCHUNK:
            copies.append(start_chunk(ci + 1))
        buf = bufs[ci % 2]

        @plsc.parallel_loop(0, CHUNK_ROWS, carry=accs, unroll=8)
        def accs_out(i, a, buf=buf):
            return tuple(a[j] + buf[pl.ds(i * E + j * LANES, LANES)]
                         for j in range(ECH))
        accs = accs_out

    for j in range(ECH):
        col_v[pl.ds(j * LANES, LANES)] = accs[j]

    plsc.subcore_barrier()
    pltpu.sync_copy(col_v, shared_col.at[idx_v], add=True)
    plsc.subcore_barrier()
    pltpu.sync_copy(shared_col, col_v)

    sel_cp.wait()

    @plsc.parallel_loop(0, SEL_PER_TILE // LANES,
                        carry=jnp.zeros((LANES,), jnp.float32), unroll=8)
    def acc(i, a):
        idx = sel_v[pl.ds(i * LANES, LANES)]
        return a + plsc.load_gather(col_v, [idx])

    vec_v[...] = acc
    pltpu.sync_copy(vec_v, shared_acc.at[iota16], add=True)
    plsc.subcore_barrier()

    @pl.when(s == 0)
    def _finish():
        pltpu.sync_copy(shared_acc, vec_v)
        total = jnp.sum(vec_v[...]) * SCALE
        vec_v[...] = jnp.full((LANES,), total, jnp.float32)
        pltpu.sync_copy(vec_v, out_hbm.at[c])


def kernel(router_probs, selected_experts):
    sel_flat = selected_experts.astype(jnp.int32).reshape(-1)
    out = _lb_loss_kernel(router_probs.reshape(-1), sel_flat)
    return out[0, 0] + out[1, 0]


# speedup vs baseline: 1.0527x; 1.0527x over previous
import functools

import jax
import jax.numpy as jnp
from jax import lax
from jax.experimental import pallas as pl
from jax.experimental.pallas import tpu as pltpu
from jax.experimental.pallas import tpu_sc as plsc

N = 16384
E = 64
K = 2
NS = 16
LANES = 16
ROWS_PER_TILE = N // NS
SEL_PER_TILE = (N * K) // NS
SCALE = float(E) / (float(N) * K * N)
ECH = E // LANES
NCHUNK = 4
CHUNK_ROWS = ROWS_PER_TILE // NCHUNK


_mesh = plsc.VectorSubcoreMesh(
    core_axis_name="c", subcore_axis_name="s", num_cores=1, num_subcores=NS
)


@functools.partial(
    pl.kernel,
    out_type=jax.ShapeDtypeStruct((LANES,), jnp.float32),
    mesh=_mesh,
    compiler_params=pltpu.CompilerParams(needs_layout_passes=False),
    scratch_types=[
        pltpu.VMEM((CHUNK_ROWS * E,), jnp.float32),
        pltpu.VMEM((CHUNK_ROWS * E,), jnp.float32),
        pltpu.VMEM((SEL_PER_TILE,), jnp.int32),
        pltpu.VMEM((E,), jnp.float32),
        pltpu.VMEM((NS * E,), jnp.float32),
        pltpu.VMEM((LANES,), jnp.float32),
        pltpu.VMEM((NS * LANES,), jnp.float32),
        pltpu.VMEM_SHARED((NS * E,), jnp.float32),
        pltpu.VMEM_SHARED((NS * LANES,), jnp.float32),
        pltpu.SemaphoreType.DMA,
        pltpu.SemaphoreType.DMA,
        pltpu.SemaphoreType.DMA,
    ],
)
def _lb_loss_kernel(probs_hbm, sel_hbm, out_hbm,
                    buf_a, buf_b, sel_v, col_v, colall_v, vec_v, accall_v,
                    shared_col, shared_acc, sem_a, sem_b, sem_sel):
    s = lax.axis_index("s")

    sel_cp = pltpu.async_copy(
        sel_hbm.at[pl.ds(s * SEL_PER_TILE, SEL_PER_TILE)], sel_v, sem_sel)

    bufs = (buf_a, buf_b)
    sems = (sem_a, sem_b)
    row_base = s * (ROWS_PER_TILE * E)

    def start_chunk(ci):
        return pltpu.async_copy(
            probs_hbm.at[pl.ds(row_base + ci * (CHUNK_ROWS * E),
                               CHUNK_ROWS * E)],
            bufs[ci % 2], sems[ci % 2])

    copies = [start_chunk(0)]
    accs = tuple(jnp.zeros((LANES,), jnp.float32) for _ in range(ECH))
    for ci in range(NCHUNK):
        copies[ci].wait()
        if ci + 1 < NCHUNK:
            copies.append(start_chunk(ci + 1))
        buf = bufs[ci % 2]

        @plsc.parallel_loop(0, CHUNK_ROWS, carry=accs, unroll=8)
        def accs_out(i, a, buf=buf):
            return tuple(a[j] + buf[pl.ds(i * E + j * LANES, LANES)]
                         for j in range(ECH))
        accs = accs_out

    for j in range(ECH):
        col_v[pl.ds(j * LANES, LANES)] = accs[j]

    pltpu.sync_copy(col_v, shared_col.at[pl.ds(s * E, E)])
    plsc.subcore_barrier()
    pltpu.sync_copy(shared_col, colall_v)
    for j in range(ECH):
        cj = colall_v[pl.ds(j * LANES, LANES)]
        for t in range(1, NS):
            cj = cj + colall_v[pl.ds(t * E + j * LANES, LANES)]
        col_v[pl.ds(j * LANES, LANES)] = cj

    sel_cp.wait()

    @plsc.parallel_loop(0, SEL_PER_TILE // LANES,
                        carry=jnp.zeros((LANES,), jnp.float32), unroll=8)
    def acc(i, a):
        idx = sel_v[pl.ds(i * LANES, LANES)]
        return a + plsc.load_gather(col_v, [idx])

    vec_v[...] = acc
    pltpu.sync_copy(vec_v, shared_acc.at[pl.ds(s * LANES, LANES)])
    plsc.subcore_barrier()

    @pl.when(s == 0)
    def _finish():
        pltpu.sync_copy(shared_acc, accall_v)
        tot = accall_v[pl.ds(0, LANES)]
        for t in range(1, NS):
            tot = tot + accall_v[pl.ds(t * LANES, LANES)]
        total = jnp.sum(tot) * SCALE
        vec_v[...] = jnp.full((LANES,), total, jnp.float32)
        pltpu.sync_copy(vec_v, out_hbm)


def kernel(router_probs, selected_experts):
    sel_flat = selected_experts.astype(jnp.int32).reshape(-1)
    out = _lb_loss_kernel(router_probs.reshape(-1), sel_flat)
    return out[0] * 1.0
